# baseline (speedup 1.0000x reference)
import functools

import jax
import jax.numpy as jnp
from jax import lax
from jax.experimental import pallas as pl
from jax.experimental.pallas import tpu as pltpu
from jax.experimental.pallas import tpu_sc as plsc

B = 16384
D = 16
NC = 2
NS = 16
NW = NC * NS
CH = B // NW
ICH = 128
NIC = CH // ICH

_mesh = plsc.VectorSubcoreMesh(core_axis_name="c", subcore_axis_name="s")


@functools.partial(
    pl.kernel,
    out_type=jax.ShapeDtypeStruct((B,), jnp.float32),
    mesh=_mesh,
    compiler_params=pltpu.CompilerParams(
        needs_layout_passes=False, use_tc_tiling_on_sc=False),
    scratch_types=dict(
        idx_u=pltpu.VMEM((NIC, ICH), jnp.int32),
        idx_i=pltpu.VMEM((NIC, ICH), jnp.int32),
        g_uvm=pltpu.VMEM((CH, D), jnp.float32),
        g_uvl=pltpu.VMEM((CH, D), jnp.float32),
        g_ivm=pltpu.VMEM((CH, D), jnp.float32),
        g_ivl=pltpu.VMEM((CH, D), jnp.float32),
        g_ubm=pltpu.VMEM((CH,), jnp.float32),
        g_ubl=pltpu.VMEM((CH,), jnp.float32),
        g_ibm=pltpu.VMEM((CH,), jnp.float32),
        g_ibl=pltpu.VMEM((CH,), jnp.float32),
        l_evu=pltpu.VMEM((CH, D), jnp.float32),
        l_evi=pltpu.VMEM((CH, D), jnp.float32),
        l_ebu=pltpu.VMEM((CH,), jnp.float32),
        l_ebi=pltpu.VMEM((CH,), jnp.float32),
        l_glob=pltpu.VMEM((D,), jnp.float32),
        prod=pltpu.VMEM((16, D), jnp.float32),
        intx=pltpu.VMEM((CH,), jnp.float32),
        out_v=pltpu.VMEM((CH,), jnp.float32),
        sem=pltpu.SemaphoreType.DMA,
    ),
)
def _vmf_sc(u, i, ubm, ubl, uvm, uvl, ibm, ibl, ivm, ivl, glob,
            ebu, evu, ebi, evi, out,
            idx_u, idx_i, g_uvm, g_uvl, g_ivm, g_ivl,
            g_ubm, g_ubl, g_ibm, g_ibl,
            l_evu, l_evi, l_ebu, l_ebi, l_glob, prod, intx, out_v, sem):
  wid = lax.axis_index("s") * NC + lax.axis_index("c")
  base = wid * CH

  for j in range(NIC):
    pltpu.sync_copy(u.at[pl.ds(base + j * ICH, ICH)], idx_u.at[j])
    pltpu.sync_copy(i.at[pl.ds(base + j * ICH, ICH)], idx_i.at[j])

  cps = [
      pltpu.async_copy(ebu.at[pl.ds(base, CH)], l_ebu, sem),
      pltpu.async_copy(ebi.at[pl.ds(base, CH)], l_ebi, sem),
      pltpu.async_copy(evu.at[pl.ds(base, CH)], l_evu, sem),
      pltpu.async_copy(evi.at[pl.ds(base, CH)], l_evi, sem),
      pltpu.async_copy(glob, l_glob, sem),
  ]
  for j in range(NIC):
    sl = pl.ds(j * ICH, ICH)
    iu = idx_u.at[j]
    ii = idx_i.at[j]
    cps += [
        pltpu.async_copy(uvm.at[iu], g_uvm.at[sl], sem),
        pltpu.async_copy(uvl.at[iu], g_uvl.at[sl], sem),
        pltpu.async_copy(ivm.at[ii], g_ivm.at[sl], sem),
        pltpu.async_copy(ivl.at[ii], g_ivl.at[sl], sem),
        pltpu.async_copy(ubm.at[iu], g_ubm.at[sl], sem),
        pltpu.async_copy(ubl.at[iu], g_ubl.at[sl], sem),
        pltpu.async_copy(ibm.at[ii], g_ibm.at[sl], sem),
        pltpu.async_copy(ibl.at[ii], g_ibl.at[sl], sem),
    ]
  for c in cps:
    c.wait()

  g = l_glob[...]
  lane = lax.broadcasted_iota(jnp.int32, (D,), 0)

  def group_body(k, carry):
    r0 = k * 16
    for r16 in range(16):
      r = r0 + r16
      vu = g_uvm[r, :] + jnp.exp(0.5 * g_uvl[r, :]) * l_evu[r, :]
      vi = g_ivm[r, :] + jnp.exp(0.5 * g_ivl[r, :]) * l_evi[r, :]
      plsc.store_scatter(prod, [lane, jnp.full((D,), r16, jnp.int32)],
                         vu * vi)
    acc = prod[0, :]
    for d in range(1, 16):
      acc = acc + prod[d, :]
    sl = pl.ds(r0, 16)
    bu = g_ubm[sl] + jnp.exp(0.5 * g_ubl[sl]) * l_ebu[sl]
    bi = g_ibm[sl] + jnp.exp(0.5 * g_ibl[sl]) * l_ebi[sl]
    out_v[sl] = bu + bi + g + acc
    return carry

  lax.fori_loop(0, CH // 16, group_body, 0)

  pltpu.sync_copy(out_v, out.at[pl.ds(base, CH)])


def kernel(u, i, user_bias_mu, user_bias_lv, user_vect_mu, user_vect_lv,
           item_bias_mu, item_bias_lv, item_vect_mu, item_vect_lv,
           glob_bias, eps_bu, eps_vu, eps_bi, eps_vi):
  return _vmf_sc(
      u, i,
      user_bias_mu.reshape(-1), user_bias_lv.reshape(-1),
      user_vect_mu, user_vect_lv,
      item_bias_mu.reshape(-1), item_bias_lv.reshape(-1),
      item_vect_mu, item_vect_lv,
      jnp.broadcast_to(glob_bias.reshape(1), (D,)),
      eps_bu, eps_vu, eps_bi, eps_vi)
